# baseline (speedup 1.0000x reference)
import functools

import jax
import jax.numpy as jnp
from jax import lax
from jax.experimental import pallas as pl
from jax.experimental.pallas import tpu as pltpu
from jax.experimental.pallas import tpu_sc as plsc

D_MODEL = 768
FF = 3072
N_EXPERT = 8
T = 2048

BLK = 256
NB = T * 2 // BLK + (N_EXPERT - 1)
S = NB * BLK
F_SPLIT = 2
F_TILE = FF // F_SPLIT


def _gate_kernel(x_ref, wg_ref, wt_ref):
    s = lax.dot_general(x_ref[...], wg_ref[...], (((1,), (1,)), ((), ())),
                        preferred_element_type=jnp.float32)
    st = lax.transpose(s, (1, 0))
    E = st.shape[0]
    iota = lax.broadcasted_iota(jnp.int32, st.shape, 0)
    m1 = jnp.max(st, axis=0, keepdims=True)
    i1 = jnp.min(jnp.where(st == m1, iota, E), axis=0, keepdims=True)
    s2 = jnp.where(iota == i1, -jnp.inf, st)
    m2 = jnp.max(s2, axis=0, keepdims=True)
    i2 = jnp.min(jnp.where(s2 == m2, iota, E), axis=0, keepdims=True)
    z = jnp.exp(m2 - m1)
    p1 = 1.0 / (1.0 + z)
    p2 = 1.0 - p1
    wt_ref[...] = jnp.where(iota == i1, p1, 0.0) + jnp.where(iota == i2, p2, 0.0)


def _gate(x_flat, Wg):
    return pl.pallas_call(
        _gate_kernel,
        out_shape=jax.ShapeDtypeStruct((N_EXPERT, T), jnp.float32),
    )(x_flat, Wg)


def _lane_shuffle(v, idx):
    return lax.gather(
        v, idx[:, None],
        lax.GatherDimensionNumbers(offset_dims=(), collapsed_slice_dims=(0,),
                                   start_index_map=(0,)),
        (1,), mode=lax.GatherScatterMode.PROMISE_IN_BOUNDS)


def _route_body(wt_hbm, gidx_hbm, sprob_hbm, binfo_hbm,
                wall, tokbuf, probbuf, binfo_v):
    cid = lax.axis_index("c")
    sid = lax.axis_index("s")
    iota16 = lax.iota(jnp.int32, 16)

    @pl.when(jnp.logical_and(cid == 0, sid < N_EXPERT))
    def _worker():
        pltpu.sync_copy(wt_hbm, wall)

        counts = []
        for j in range(N_EXPERT):
            def cbody(c, acc, j=j):
                wv = wall[pl.ds(j * T + c * 16, 16)]
                return acc + jnp.where(wv > 0.0, 1, 0)

            acc = lax.fori_loop(0, T // 16, cbody, jnp.zeros((16,), jnp.int32))
            for sh in (8, 4, 2, 1):
                acc = acc + _lane_shuffle(acc, jnp.bitwise_xor(iota16, sh))
            counts.append(acc[0])

        start = jnp.int32(0)
        my_start = jnp.int32(0)
        my_cnt = jnp.int32(0)
        pstarts = []
        actives = []
        for j in range(N_EXPERT):
            cj = counts[j]
            pstarts.append(start)
            actives.append(cj > 0)
            my_start = jnp.where(sid == j, start, my_start)
            my_cnt = jnp.where(sid == j, cj, my_cnt)
            start = start + ((cj + BLK - 1) // BLK) * BLK
        total = start

        def zbody(i, _):
            tokbuf[pl.ds(i * 16, 16)] = jnp.zeros((16,), jnp.int32)
            probbuf[pl.ds(i * 16, 16)] = jnp.zeros((16,), jnp.float32)
            return 0

        lax.fori_loop(0, (T + 16) // 16, zbody, 0)

        row_base = pl.multiple_of(sid * T, T)

        def ebody(c, off):
            wv = wall[pl.ds(row_base + c * 16, 16)]
            for l in range(16):
                wl = wv[l]
                tokbuf[pl.ds(off, 16)] = jnp.full((16,), c * 16 + l, jnp.int32)
                probbuf[pl.ds(off, 16)] = jnp.full((16,), wl, jnp.float32)
                off = off + jnp.where(wl > 0.0, 1, 0)
            return off

        off = lax.fori_loop(0, T // 16, ebody, jnp.int32(0))
        tokbuf[pl.ds(off, 16)] = jnp.zeros((16,), jnp.int32)
        probbuf[pl.ds(off, 16)] = jnp.zeros((16,), jnp.float32)

        def obody(k, _):
            dst = pl.multiple_of(my_start + k * BLK, BLK)
            pltpu.sync_copy(tokbuf.at[pl.ds(k * BLK, BLK)],
                            gidx_hbm.at[pl.ds(dst, BLK)])
            pltpu.sync_copy(probbuf.at[pl.ds(k * BLK, BLK)],
                            sprob_hbm.at[pl.ds(dst, BLK)])
            return 0

        lax.fori_loop(0, (my_cnt + BLK - 1) // BLK, obody, 0)

        @pl.when(sid == 0)
        def _binfo():
            for k in range(2):
                slot0 = (iota16 + k * 16) * BLK
                be = jnp.zeros((16,), jnp.int32)
                for j in range(N_EXPERT):
                    aj = jnp.where(actives[j], 1, 0)
                    condv = jnp.where(slot0 >= pstarts[j], aj, 0)
                    be = jnp.where(condv > 0, j, be)
                binfo_v[pl.ds(k * 16, 16)] = be
            binfo_v[pl.ds(32, 16)] = jnp.full((16,), total, jnp.int32)
            for k in range(3, 8):
                binfo_v[pl.ds(k * 16, 16)] = jnp.zeros((16,), jnp.int32)
            pltpu.sync_copy(binfo_v, binfo_hbm)


def _route(wt):
    mesh = plsc.VectorSubcoreMesh(core_axis_name="c", subcore_axis_name="s")
    return pl.kernel(
        _route_body,
        out_type=[
            jax.ShapeDtypeStruct((S,), jnp.int32),
            jax.ShapeDtypeStruct((S,), jnp.float32),
            jax.ShapeDtypeStruct((128,), jnp.int32),
        ],
        mesh=mesh,
        scratch_types=[
            pltpu.VMEM((N_EXPERT * T,), jnp.float32),
            pltpu.VMEM((T + 16,), jnp.int32),
            pltpu.VMEM((T + 16,), jnp.float32),
            pltpu.VMEM((128,), jnp.int32),
        ],
    )(wt)


def _ffn_kernel(be_ref, tot_ref, x_ref, gidx_ref, sprob_ref, w1_ref, w2_ref,
                y_ref):
    nb = pl.program_id(0)

    @pl.when(nb == 0)
    def _init():
        y_ref[...] = jnp.zeros_like(y_ref)

    @pl.when(nb * BLK < tot_ref[0])
    def _active():
        gi = gidx_ref[0]
        iota = lax.broadcasted_iota(jnp.int32, (T, BLK), 0)
        oh = jnp.where(iota == gi, 1.0, 0.0)
        xg = lax.dot_general(oh, x_ref[...], (((0,), (0,)), ((), ())),
                             preferred_element_type=jnp.float32)
        h = lax.dot_general(xg, w1_ref[0], (((1,), (1,)), ((), ())),
                            preferred_element_type=jnp.float32)
        h = h * (1.0 / (1.0 + jnp.exp(-h)))
        o = lax.dot_general(h, w2_ref[0], (((1,), (1,)), ((), ())),
                            preferred_element_type=jnp.float32)
        ohw = oh * sprob_ref[0]
        y_ref[...] += lax.dot_general(
            ohw, o, (((1,), (0,)), ((), ())),
            preferred_element_type=jnp.float32)


def _ffn(be, tot, x_flat, gidx, sprob, W1, W2):
    grid_spec = pltpu.PrefetchScalarGridSpec(
        num_scalar_prefetch=2,
        grid=(NB,),
        in_specs=[
            pl.BlockSpec((T, D_MODEL), lambda nb, be, tot: (0, 0)),
            pl.BlockSpec((1, 1, BLK), lambda nb, be, tot: (nb, 0, 0)),
            pl.BlockSpec((1, 1, BLK), lambda nb, be, tot: (nb, 0, 0)),
            pl.BlockSpec((1, FF, D_MODEL), lambda nb, be, tot: (be[nb], 0, 0)),
            pl.BlockSpec((1, D_MODEL, FF), lambda nb, be, tot: (be[nb], 0, 0)),
        ],
        out_specs=pl.BlockSpec((T, D_MODEL), lambda nb, be, tot: (0, 0)),
    )
    return pl.pallas_call(
        _ffn_kernel,
        grid_spec=grid_spec,
        out_shape=jax.ShapeDtypeStruct((T, D_MODEL), jnp.float32),
        compiler_params=pltpu.CompilerParams(
            dimension_semantics=("arbitrary",)),
    )(be, tot, x_flat, gidx, sprob, W1, W2)


def kernel(x, Wg, W1, W2):
    B, Tx, C = x.shape
    x_flat = x.reshape(Tx, C)
    wt = _gate(x_flat, Wg)
    gidx, sprob, binfo = _route(wt.reshape(N_EXPERT * T))
    be = binfo[:32]
    tot = binfo[32:48]
    y = _ffn(be, tot, x_flat,
             gidx.reshape(NB, 1, BLK), sprob.reshape(NB, 1, BLK), W1, W2)
    return y.reshape(B, Tx, C)


# speedup vs baseline: 1.0025x; 1.0025x over previous
import jax
import jax.numpy as jnp
from jax import lax
from jax.experimental import pallas as pl
from jax.experimental.pallas import tpu as pltpu
from jax.experimental.pallas import tpu_sc as plsc

D_MODEL = 768
FF = 3072
N_EXPERT = 8
T = 2048

BLK = 256
NB = T * 2 // BLK + (N_EXPERT - 1)
S = NB * BLK


def _gate_kernel(x_ref, wg_ref, wt_ref):
    s = lax.dot_general(x_ref[...], wg_ref[...], (((1,), (1,)), ((), ())),
                        preferred_element_type=jnp.float32)
    st = lax.transpose(s, (1, 0))
    E = st.shape[0]
    iota = lax.broadcasted_iota(jnp.int32, st.shape, 0)
    m1 = jnp.max(st, axis=0, keepdims=True)
    i1 = jnp.min(jnp.where(st == m1, iota, E), axis=0, keepdims=True)
    s2 = jnp.where(iota == i1, -jnp.inf, st)
    m2 = jnp.max(s2, axis=0, keepdims=True)
    i2 = jnp.min(jnp.where(s2 == m2, iota, E), axis=0, keepdims=True)
    z = jnp.exp(m2 - m1)
    p1 = 1.0 / (1.0 + z)
    p2 = 1.0 - p1
    wt_ref[...] = jnp.where(iota == i1, p1, 0.0) + jnp.where(iota == i2, p2, 0.0)


def _gate(x_flat, Wg):
    return pl.pallas_call(
        _gate_kernel,
        out_shape=jax.ShapeDtypeStruct((N_EXPERT, T), jnp.float32),
    )(x_flat, Wg)


def _lane_shuffle(v, idx):
    return lax.gather(
        v, idx[:, None],
        lax.GatherDimensionNumbers(offset_dims=(), collapsed_slice_dims=(0,),
                                   start_index_map=(0,)),
        (1,), mode=lax.GatherScatterMode.PROMISE_IN_BOUNDS)


def _route_body(wt_hbm, gidx_hbm, sprob_hbm, binfo_hbm,
                wall, tokbuf, probbuf, binfo_v):
    cid = lax.axis_index("c")
    sid = lax.axis_index("s")
    iota16 = lax.iota(jnp.int32, 16)

    @pl.when(jnp.logical_and(cid == 0, sid < N_EXPERT))
    def _worker():
        pltpu.sync_copy(wt_hbm, wall)

        counts = []
        for j in range(N_EXPERT):
            def cbody(c, acc, j=j):
                wv = wall[pl.ds(j * T + c * 16, 16)]
                return acc + jnp.where(wv > 0.0, 1, 0)

            acc = lax.fori_loop(0, T // 16, cbody, jnp.zeros((16,), jnp.int32))
            for sh in (8, 4, 2, 1):
                acc = acc + _lane_shuffle(acc, jnp.bitwise_xor(iota16, sh))
            counts.append(acc[0])

        start = jnp.int32(0)
        my_start = jnp.int32(0)
        my_cnt = jnp.int32(0)
        pstarts = []
        actives = []
        for j in range(N_EXPERT):
            cj = counts[j]
            pstarts.append(start)
            actives.append(cj > 0)
            my_start = jnp.where(sid == j, start, my_start)
            my_cnt = jnp.where(sid == j, cj, my_cnt)
            start = start + ((cj + BLK - 1) // BLK) * BLK
        total = start

        def zbody(i, _):
            tokbuf[pl.ds(i * 16, 16)] = jnp.zeros((16,), jnp.int32)
            probbuf[pl.ds(i * 16, 16)] = jnp.zeros((16,), jnp.float32)
            return 0

        lax.fori_loop(0, (T + 16) // 16, zbody, 0)

        row_base = pl.multiple_of(sid * T, T)

        def ebody(c, off):
            wv = wall[pl.ds(row_base + c * 16, 16)]
            for l in range(16):
                wl = wv[l]
                tokbuf[pl.ds(off, 16)] = jnp.full((16,), c * 16 + l, jnp.int32)
                probbuf[pl.ds(off, 16)] = jnp.full((16,), wl, jnp.float32)
                off = off + jnp.where(wl > 0.0, 1, 0)
            return off

        off = lax.fori_loop(0, T // 16, ebody, jnp.int32(0))
        tokbuf[pl.ds(off, 16)] = jnp.zeros((16,), jnp.int32)
        probbuf[pl.ds(off, 16)] = jnp.zeros((16,), jnp.float32)

        def obody(k, _):
            dst = pl.multiple_of(my_start + k * BLK, BLK)
            pltpu.sync_copy(tokbuf.at[pl.ds(k * BLK, BLK)],
                            gidx_hbm.at[pl.ds(dst, BLK)])
            pltpu.sync_copy(probbuf.at[pl.ds(k * BLK, BLK)],
                            sprob_hbm.at[pl.ds(dst, BLK)])
            return 0

        lax.fori_loop(0, (my_cnt + BLK - 1) // BLK, obody, 0)

        @pl.when(sid == 0)
        def _binfo():
            for k in range(2):
                slot0 = (iota16 + k * 16) * BLK
                be = jnp.zeros((16,), jnp.int32)
                for j in range(N_EXPERT):
                    aj = jnp.where(actives[j], 1, 0)
                    condv = jnp.where(slot0 >= pstarts[j], aj, 0)
                    be = jnp.where(condv > 0, j, be)
                binfo_v[pl.ds(k * 16, 16)] = be
            binfo_v[pl.ds(32, 16)] = jnp.full((16,), total, jnp.int32)
            for k in range(3, 8):
                binfo_v[pl.ds(k * 16, 16)] = jnp.zeros((16,), jnp.int32)
            pltpu.sync_copy(binfo_v, binfo_hbm)


def _route(wt):
    mesh = plsc.VectorSubcoreMesh(core_axis_name="c", subcore_axis_name="s")
    return pl.kernel(
        _route_body,
        out_type=[
            jax.ShapeDtypeStruct((S,), jnp.int32),
            jax.ShapeDtypeStruct((S,), jnp.float32),
            jax.ShapeDtypeStruct((128,), jnp.int32),
        ],
        mesh=mesh,
        scratch_types=[
            pltpu.VMEM((N_EXPERT * T,), jnp.float32),
            pltpu.VMEM((T + 16,), jnp.int32),
            pltpu.VMEM((T + 16,), jnp.float32),
            pltpu.VMEM((128,), jnp.int32),
        ],
    )(wt)


def _ffn_kernel(be_ref, tot_ref, x_ref, gidx_ref, sprob_ref, w1_ref, w2_ref,
                y_ref):
    nb = pl.program_id(0)

    @pl.when(nb == 0)
    def _init():
        y_ref[...] = jnp.zeros_like(y_ref)

    @pl.when(nb * BLK < tot_ref[0])
    def _active():
        gi = gidx_ref[0]
        iota = lax.broadcasted_iota(jnp.int32, (T, BLK), 0)
        oh = jnp.where(iota == gi, 1.0, 0.0)
        xg = lax.dot_general(oh, x_ref[...], (((0,), (0,)), ((), ())),
                             preferred_element_type=jnp.float32)
        h = lax.dot_general(xg, w1_ref[0], (((1,), (1,)), ((), ())),
                            preferred_element_type=jnp.float32)
        h = h * (1.0 / (1.0 + jnp.exp(-h)))
        o = lax.dot_general(h, w2_ref[0], (((1,), (1,)), ((), ())),
                            preferred_element_type=jnp.float32)
        ohw = oh * sprob_ref[0]
        y_ref[...] += lax.dot_general(
            ohw, o, (((1,), (0,)), ((), ())),
            preferred_element_type=jnp.float32)


def _ffn(be, tot, x_flat, gidx, sprob, W1, W2):
    grid_spec = pltpu.PrefetchScalarGridSpec(
        num_scalar_prefetch=2,
        grid=(NB,),
        in_specs=[
            pl.BlockSpec((T, D_MODEL), lambda nb, be, tot: (0, 0)),
            pl.BlockSpec((1, 1, BLK), lambda nb, be, tot: (nb, 0, 0)),
            pl.BlockSpec((1, 1, BLK), lambda nb, be, tot: (nb, 0, 0)),
            pl.BlockSpec((1, FF, D_MODEL), lambda nb, be, tot: (be[nb], 0, 0)),
            pl.BlockSpec((1, D_MODEL, FF), lambda nb, be, tot: (be[nb], 0, 0)),
        ],
        out_specs=pl.BlockSpec((T, D_MODEL), lambda nb, be, tot: (0, 0)),
    )
    return pl.pallas_call(
        _ffn_kernel,
        grid_spec=grid_spec,
        out_shape=jax.ShapeDtypeStruct((T, D_MODEL), jnp.float32),
        compiler_params=pltpu.CompilerParams(
            dimension_semantics=("arbitrary",)),
    )(be, tot, x_flat, gidx, sprob, W1, W2)


def kernel(x, Wg, W1, W2):
    B, Tx, C = x.shape
    x_flat = x.reshape(Tx, C)
    wt = _gate(x_flat, Wg)
    gidx, sprob, binfo = _route(wt.reshape(N_EXPERT * T))
    be = binfo[:32]
    tot = binfo[32:48]
    y = _ffn(be, tot, x_flat,
             gidx.reshape(NB, 1, BLK), sprob.reshape(NB, 1, BLK), W1, W2)
    return y.reshape(B, Tx, C)
